# Initial kernel scaffold; baseline (speedup 1.0000x reference)
import jax
import jax.numpy as jnp
from jax.experimental import pallas as pl


def kernel(tokens, s, idx):
    raise NotImplementedError("write your pallas kernel here")


# speedup vs baseline: 1.1478x; 1.1478x over previous
import functools

import jax
import jax.numpy as jnp
from jax import lax
from jax.experimental import pallas as pl
from jax.experimental.pallas import tpu as pltpu
from jax.experimental.pallas import tpu_sc as plsc

N_ROWS = 8192
D = 4096
N_IDX = 4096
NC = 2
NS = 16
PER_SUB = N_IDX // (NC * NS)
BINS_PER_SUB = N_ROWS // NS
ROW_BLK = 1024
W_BLK = ROW_BLK


def _sc_hist(idx_hbm, s_hbm, w_hbm, idx_v, ssel_v, stage_v, shared, sem):
    cid = lax.axis_index("c")
    sid = lax.axis_index("s")
    base = cid * (N_IDX // NC) + sid * PER_SUB

    def zero_chunk(k, _):
        stage_v[pl.ds(k * 16, 16)] = jnp.zeros((16,), jnp.float32)
        return 0
    lax.fori_loop(0, BINS_PER_SUB // 16, zero_chunk, 0)
    pltpu.sync_copy(stage_v, shared.at[pl.ds(sid * BINS_PER_SUB, BINS_PER_SUB)])
    plsc.subcore_barrier()

    pltpu.sync_copy(idx_hbm.at[pl.ds(base, PER_SUB)], idx_v)
    pltpu.async_copy(s_hbm.at[idx_v], ssel_v, sem).wait()
    pltpu.sync_copy(ssel_v, shared.at[idx_v], add=True)
    plsc.subcore_barrier()

    pltpu.sync_copy(shared.at[pl.ds(sid * BINS_PER_SUB, BINS_PER_SUB)], stage_v)
    pltpu.sync_copy(stage_v, w_hbm.at[cid, pl.ds(sid * BINS_PER_SUB, BINS_PER_SUB)])


def _mv_body(w_ref, t_ref, o_ref, dsum_ref):
    pid = pl.program_id(0)

    @pl.when(pid == 0)
    def _init():
        o_ref[...] = jnp.zeros_like(o_ref)
        dsum_ref[0] = 0.0

    wrow = w_ref[0, 0] + w_ref[1, 0]
    o_ref[...] += jax.lax.dot_general(
        wrow, t_ref[...], (((1,), (0,)), ((), ())),
        precision=jax.lax.Precision.HIGHEST,
        preferred_element_type=jnp.float32)
    dsum_ref[0] += jnp.sum(wrow)

    @pl.when(pid == pl.num_programs(0) - 1)
    def _fin():
        o_ref[...] = o_ref[...] / (dsum_ref[0] + 1e-6)


def kernel(tokens, s, idx):
    idx32 = idx.astype(jnp.int32)

    mesh = plsc.VectorSubcoreMesh(core_axis_name="c", subcore_axis_name="s")
    hist = functools.partial(
        pl.kernel,
        mesh=mesh,
        out_type=jax.ShapeDtypeStruct((NC, N_ROWS), jnp.float32),
        scratch_types=[
            pltpu.VMEM((PER_SUB,), jnp.int32),
            pltpu.VMEM((PER_SUB,), jnp.float32),
            pltpu.VMEM((BINS_PER_SUB,), jnp.float32),
            pltpu.VMEM_SHARED((N_ROWS,), jnp.float32),
            pltpu.SemaphoreType.DMA,
        ],
    )(_sc_hist)
    w = hist(idx32, s)
    w4 = w.reshape(NC, N_ROWS // W_BLK, 1, W_BLK)

    out = pl.pallas_call(
        _mv_body,
        grid=(N_ROWS // ROW_BLK,),
        in_specs=[
            pl.BlockSpec((NC, 1, 1, W_BLK), lambda i: (0, i, 0, 0)),
            pl.BlockSpec((ROW_BLK, D), lambda i: (i, 0)),
        ],
        out_specs=pl.BlockSpec((1, D), lambda i: (0, 0)),
        out_shape=jax.ShapeDtypeStruct((1, D), jnp.float32),
        scratch_shapes=[pltpu.SMEM((1,), jnp.float32)],
    )(w4, tokens)

    return out
